# baseline (speedup 1.0000x reference)
import jax
import jax.numpy as jnp
from jax import lax
from jax.experimental import pallas as pl
from jax.experimental.pallas import tpu as pltpu
from jax.experimental.pallas import tpu_sc as plsc

B, H, W, G, C, K = 4, 56, 56, 8, 16, 9
HW = H * W
NW = 32
WP = 64
NCHUNK = WP // 16
SKEW = HW + 1
D2 = 2 * K


def _dcn_body(inp_hbm, def_hbm, wt_hbm, out_hbm,
              stage, slab, def_v, wt_v, out_cm, out_pm,
              sem_in, sem_def, sem_wt, sem_out):
    cid = lax.axis_index("c")
    sid = lax.axis_index("s")
    wid = sid * 2 + cid
    b = wid // G
    g = wid - b * G

    iota = lax.iota(jnp.int32, 16)
    iota_f = iota.astype(jnp.float32)
    zero16 = jnp.zeros((16,), jnp.float32)

    pltpu.sync_copy(inp_hbm.at[b, :, pl.ds(g * C, C)], stage)

    mask9 = iota < K
    for rbuf in range(2):
        rb = jnp.full((16,), rbuf)
        for w in range(W, WP):
            wv = jnp.full((16,), w)
            plsc.store_scatter(def_v, [rb, wv, iota], zero16)
            plsc.store_scatter(def_v, [rb, wv, iota + 2], zero16)
            plsc.store_scatter(wt_v, [rb, wv, iota], zero16, mask=mask9)

    def tr_body(pc, _):
        p0 = pc * 16
        for j in range(16):
            v = plsc.load_gather(stage, [jnp.full((16,), p0 + j), iota])
            plsc.store_scatter(slab, [iota * SKEW + (p0 + j)], v)
        return 0

    lax.fori_loop(0, HW // 16, tr_body, 0)

    pltpu.async_copy(def_hbm.at[b, 0, :, g, :], def_v.at[0, pl.ds(0, W)],
                     sem_def)
    pltpu.async_copy(wt_hbm.at[b, 0, :, g, :], wt_v.at[0, pl.ds(0, W)],
                     sem_wt)

    def row_step(h, _):
        buf = lax.rem(h, 2)

        @pl.when(h + 1 < H)
        def _start_next():
            nbuf = lax.rem(h + 1, 2)
            pltpu.async_copy(def_hbm.at[b, h + 1, :, g, :],
                             def_v.at[nbuf, pl.ds(0, W)], sem_def)
            pltpu.async_copy(wt_hbm.at[b, h + 1, :, g, :],
                             wt_v.at[nbuf, pl.ds(0, W)], sem_wt)

        pltpu.make_async_copy(def_hbm.at[b, h, :, g, :],
                              def_v.at[buf, pl.ds(0, W)], sem_def).wait()
        pltpu.make_async_copy(wt_hbm.at[b, h, :, g, :],
                              wt_v.at[buf, pl.ds(0, W)], sem_wt).wait()

        @pl.when(h >= 2)
        def _drain_out():
            pltpu.make_async_copy(out_pm.at[buf],
                                  out_hbm.at[b, h - 2, :, g, :],
                                  sem_out).wait()

        hf = h.astype(jnp.float32)
        bvec = jnp.full((16,), buf)

        def chunk_body(cb, _):
            cb16 = cb * 16
            wvec_i = cb16 + iota
            wvec = wvec_i.astype(jnp.float32)
            accs = tuple(zero16 for _ in range(C))

            for k in range(K):
                dxv = plsc.load_gather(def_v, [bvec, wvec_i,
                                               jnp.full((16,), 2 * k)])
                dyv = plsc.load_gather(def_v, [bvec, wvec_i,
                                               jnp.full((16,), 2 * k + 1)])
                wkv = plsc.load_gather(wt_v, [bvec, wvec_i,
                                              jnp.full((16,), k)])
                x = dxv + wvec
                y = dyv + hf
                xi = jnp.clip(x, -4.0, 60.0).astype(jnp.int32)
                yi = jnp.clip(y, -4.0, 60.0).astype(jnp.int32)
                fx = xi.astype(jnp.float32)
                fy = yi.astype(jnp.float32)
                tx = x - fx
                ty = y - fy
                ox = 1.0 - tx
                oy = 1.0 - ty
                mx0 = (xi >= 0) & (xi < W)
                mx1 = (xi >= -1) & (xi < W - 1)
                my0 = (yi >= 0) & (yi < H)
                my1 = (yi >= -1) & (yi < H - 1)
                wtl = jnp.where(mx0 & my0, wkv * (ox * oy), zero16)
                wtr = jnp.where(mx1 & my0, wkv * (tx * oy), zero16)
                wbl = jnp.where(mx0 & my1, wkv * (ox * ty), zero16)
                wbr = jnp.where(mx1 & my1, wkv * (tx * ty), zero16)
                x0 = jnp.clip(xi, 0, W - 1)
                x1 = jnp.clip(xi + 1, 0, W - 1)
                y0 = jnp.clip(yi, 0, H - 1) * W
                y1 = jnp.clip(yi + 1, 0, H - 1) * W
                a00 = y0 + x0
                a10 = y0 + x1
                a01 = y1 + x0
                a11 = y1 + x1
                new = []
                for c in range(C):
                    v00 = plsc.load_gather(slab, [a00 + c * SKEW])
                    v10 = plsc.load_gather(slab, [a10 + c * SKEW])
                    v01 = plsc.load_gather(slab, [a01 + c * SKEW])
                    v11 = plsc.load_gather(slab, [a11 + c * SKEW])
                    new.append(accs[c] + (v00 * wtl + v10 * wtr
                                          + v01 * wbl + v11 * wbr))
                accs = tuple(new)

            for c in range(C):
                plsc.store_scatter(out_cm, [bvec, jnp.full((16,), c),
                                            wvec_i], accs[c])
            return 0

        lax.fori_loop(0, NCHUNK, chunk_body, 0)

        def out_tr(p, _):
            v = plsc.load_gather(out_cm, [bvec, iota, jnp.full((16,), p)])
            plsc.store_scatter(out_pm, [bvec, jnp.full((16,), p), iota], v)
            return 0

        lax.fori_loop(0, W, out_tr, 0)

        pltpu.async_copy(out_pm.at[buf], out_hbm.at[b, h, :, g, :], sem_out)
        return 0

    lax.fori_loop(0, H, row_step, 0)

    pltpu.make_async_copy(out_pm.at[0], out_hbm.at[b, H - 2, :, g, :],
                          sem_out).wait()
    pltpu.make_async_copy(out_pm.at[1], out_hbm.at[b, H - 1, :, g, :],
                          sem_out).wait()


@jax.jit
def kernel(input, deformable, weights):
    inp_r = input.reshape(B, HW, G * C)
    def_r = deformable.reshape(B, H, W, G, D2)

    mesh = plsc.VectorSubcoreMesh(core_axis_name="c", subcore_axis_name="s",
                                  num_cores=2, num_subcores=16)
    run = pl.kernel(
        _dcn_body,
        out_type=jax.ShapeDtypeStruct((B, H, W, G, C), jnp.float32),
        mesh=mesh,
        scratch_types=[
            pltpu.VMEM((HW, C), jnp.float32),
            pltpu.VMEM((C * SKEW,), jnp.float32),
            pltpu.VMEM((2, WP, D2), jnp.float32),
            pltpu.VMEM((2, WP, K), jnp.float32),
            pltpu.VMEM((2, C, WP + 1), jnp.float32),
            pltpu.VMEM((2, W, C), jnp.float32),
            pltpu.SemaphoreType.DMA,
            pltpu.SemaphoreType.DMA,
            pltpu.SemaphoreType.DMA,
            pltpu.SemaphoreType.DMA,
        ],
        compiler_params=pltpu.CompilerParams(needs_layout_passes=False,
                                             use_tc_tiling_on_sc=False),
    )
    return run(inp_r, def_r, weights)


# speedup vs baseline: 1.2723x; 1.2723x over previous
import jax
import jax.numpy as jnp
from jax import lax
from jax.experimental import pallas as pl
from jax.experimental.pallas import tpu as pltpu
from jax.experimental.pallas import tpu_sc as plsc

B, H, W, G, C, K = 4, 56, 56, 8, 16, 9
NW = 32
WP = 64
NCHUNK = WP // 16
PAR_ROW = 3 * K * WP
PAR_PAD = 1792
OUT_ROW = C * W
INP_WORDS = H * W * C
PAR_OFF = INP_WORDS


def _dcn_body(prep_hbm, out_hbm, slab, par_v, out_v,
              sem_in, sem_par, sem_out):
    cid = lax.axis_index("c")
    sid = lax.axis_index("s")
    wid = sid * 2 + cid

    pltpu.sync_copy(prep_hbm.at[wid, pl.ds(0, INP_WORDS)], slab)

    pltpu.async_copy(prep_hbm.at[wid, pl.ds(PAR_OFF, PAR_PAD)],
                     par_v.at[pl.ds(0, PAR_PAD)], sem_par)

    iota = lax.iota(jnp.int32, 16)
    iota_f = iota.astype(jnp.float32)

    def row_step(h, _):
        buf = lax.rem(h, 2)
        pbase = buf * PAR_PAD
        obase = buf * OUT_ROW

        @pl.when(h + 1 < H)
        def _start_next_par():
            nbase = lax.rem(h + 1, 2) * PAR_PAD
            pltpu.async_copy(
                prep_hbm.at[wid, pl.ds(PAR_OFF + (h + 1) * PAR_PAD, PAR_PAD)],
                par_v.at[pl.ds(nbase, PAR_PAD)], sem_par)

        pltpu.make_async_copy(
            prep_hbm.at[wid, pl.ds(PAR_OFF + h * PAR_PAD, PAR_PAD)],
            par_v.at[pl.ds(pbase, PAR_PAD)], sem_par).wait()

        @pl.when(h >= 2)
        def _drain_out():
            pltpu.make_async_copy(out_v.at[pl.ds(obase, OUT_ROW)],
                                  out_hbm.at[wid, h - 2], sem_out).wait()

        hf = h.astype(jnp.float32)

        def chunk_body(cb, _):
            cb16 = cb * 16
            wvec = iota_f + cb16.astype(jnp.float32)
            accs = tuple(jnp.zeros((16,), jnp.float32) for _ in range(C))

            for k in range(K):
                pidx = pbase + (k * WP + cb16) + iota
                dxv = plsc.load_gather(par_v, [pidx])
                dyv = plsc.load_gather(par_v, [pidx + K * WP])
                wkv = plsc.load_gather(par_v, [pidx + 2 * K * WP])
                x = dxv + wvec
                y = dyv + hf
                xi = jnp.clip(x, -4.0, 60.0).astype(jnp.int32)
                yi = jnp.clip(y, -4.0, 60.0).astype(jnp.int32)
                fx = xi.astype(jnp.float32)
                fy = yi.astype(jnp.float32)
                tx = x - fx
                ty = y - fy
                ox = 1.0 - tx
                oy = 1.0 - ty
                mx0 = (xi >= 0) & (xi < W)
                mx1 = (xi >= -1) & (xi < W - 1)
                my0 = (yi >= 0) & (yi < H)
                my1 = (yi >= -1) & (yi < H - 1)
                zero = jnp.zeros((16,), jnp.float32)
                wtl = jnp.where(mx0 & my0, wkv * (ox * oy), zero)
                wtr = jnp.where(mx1 & my0, wkv * (tx * oy), zero)
                wbl = jnp.where(mx0 & my1, wkv * (ox * ty), zero)
                wbr = jnp.where(mx1 & my1, wkv * (tx * ty), zero)
                x0 = jnp.clip(xi, 0, W - 1)
                x1 = jnp.clip(xi + 1, 0, W - 1)
                y0 = jnp.clip(yi, 0, H - 1) * W
                y1 = jnp.clip(yi + 1, 0, H - 1) * W
                a00 = y0 + x0
                a10 = y0 + x1
                a01 = y1 + x0
                a11 = y1 + x1
                new = []
                for c in range(C):
                    v00 = plsc.load_gather(slab, [a00 + c * (H * W)])
                    v10 = plsc.load_gather(slab, [a10 + c * (H * W)])
                    v01 = plsc.load_gather(slab, [a01 + c * (H * W)])
                    v11 = plsc.load_gather(slab, [a11 + c * (H * W)])
                    new.append(accs[c] + (v00 * wtl + v10 * wtr
                                          + v01 * wbl + v11 * wbr))
                accs = tuple(new)

            oidx = obase + cb16 + iota
            omask = (cb16 + iota) < W
            for c in range(C):
                plsc.store_scatter(out_v, [oidx + c * W], accs[c], mask=omask)
            return 0

        lax.fori_loop(0, NCHUNK, chunk_body, 0)

        pltpu.async_copy(out_v.at[pl.ds(obase, OUT_ROW)],
                         out_hbm.at[wid, h], sem_out)
        return 0

    lax.fori_loop(0, H, row_step, 0)

    pltpu.make_async_copy(out_v.at[pl.ds(0, OUT_ROW)],
                          out_hbm.at[wid, H - 2], sem_out).wait()
    pltpu.make_async_copy(out_v.at[pl.ds(OUT_ROW, OUT_ROW)],
                          out_hbm.at[wid, H - 1], sem_out).wait()


@jax.jit
def kernel(input, deformable, weights):
    inp_t = input.transpose(0, 3, 4, 1, 2).reshape(NW, INP_WORDS)
    dx = deformable[..., 0].transpose(0, 3, 1, 4, 2)
    dy = deformable[..., 1].transpose(0, 3, 1, 4, 2)
    wk = weights.transpose(0, 3, 1, 4, 2)
    par = jnp.concatenate([dx, dy, wk], axis=3)
    par = jnp.pad(par, ((0, 0), (0, 0), (0, 0), (0, 0), (0, WP - W)))
    par = par.reshape(NW, H, PAR_ROW)
    par = jnp.pad(par, ((0, 0), (0, 0), (0, PAR_PAD - PAR_ROW)))
    prep = jnp.concatenate([inp_t, par.reshape(NW, H * PAR_PAD)], axis=1)

    mesh = plsc.VectorSubcoreMesh(core_axis_name="c", subcore_axis_name="s",
                                  num_cores=2, num_subcores=16)
    run = pl.kernel(
        _dcn_body,
        out_type=jax.ShapeDtypeStruct((NW, H, OUT_ROW), jnp.float32),
        mesh=mesh,
        scratch_types=[
            pltpu.VMEM((H * W * C,), jnp.float32),
            pltpu.VMEM((2 * PAR_PAD,), jnp.float32),
            pltpu.VMEM((2 * OUT_ROW,), jnp.float32),
            pltpu.SemaphoreType.DMA,
            pltpu.SemaphoreType.DMA,
            pltpu.SemaphoreType.DMA,
        ],
        compiler_params=pltpu.CompilerParams(needs_layout_passes=False),
    )
    out = run(prep)
    out = out.reshape(B, G, H, C, W)
    return out.transpose(0, 2, 4, 1, 3)


# speedup vs baseline: 1.3995x; 1.1000x over previous
import jax
import jax.numpy as jnp
from jax import lax
from jax.experimental import pallas as pl
from jax.experimental.pallas import tpu as pltpu
from jax.experimental.pallas import tpu_sc as plsc

B, H, W, G, C, K = 4, 56, 56, 8, 16, 9
NW = 32
WP = 64
NCHUNK = WP // 16
PAR_ROW = 3 * K * WP
PAR_PAD = 1792
OUT_ROW = C * W
INP_WORDS = H * W * C
PAR_OFF = INP_WORDS


def _dcn_body(inp_hbm, par_hbm, out_hbm, slab, par_v, out_v,
              sem_in, sem_par, sem_out):
    cid = lax.axis_index("c")
    sid = lax.axis_index("s")
    wid = sid * 2 + cid

    pltpu.sync_copy(inp_hbm.at[wid], slab)

    pltpu.async_copy(par_hbm.at[wid, 0], par_v.at[pl.ds(0, PAR_PAD)], sem_par)

    iota = lax.iota(jnp.int32, 16)
    iota_f = iota.astype(jnp.float32)

    def row_step(h, _):
        buf = lax.rem(h, 2)
        pbase = buf * PAR_PAD
        obase = buf * OUT_ROW

        @pl.when(h + 1 < H)
        def _start_next_par():
            nbase = lax.rem(h + 1, 2) * PAR_PAD
            pltpu.async_copy(par_hbm.at[wid, h + 1],
                             par_v.at[pl.ds(nbase, PAR_PAD)], sem_par)

        pltpu.make_async_copy(par_hbm.at[wid, h],
                              par_v.at[pl.ds(pbase, PAR_PAD)], sem_par).wait()

        @pl.when(h >= 2)
        def _drain_out():
            pltpu.make_async_copy(out_v.at[pl.ds(obase, OUT_ROW)],
                                  out_hbm.at[wid, h - 2], sem_out).wait()

        hf = h.astype(jnp.float32)

        def chunk_body(cb, _):
            cb16 = cb * 16
            wvec = iota_f + cb16.astype(jnp.float32)
            accs = tuple(jnp.zeros((16,), jnp.float32) for _ in range(C))

            for k in range(K):
                pidx = pbase + (k * WP + cb16) + iota
                dxv = plsc.load_gather(par_v, [pidx])
                dyv = plsc.load_gather(par_v, [pidx + K * WP])
                wkv = plsc.load_gather(par_v, [pidx + 2 * K * WP])
                x = dxv + wvec
                y = dyv + hf
                xi = jnp.clip(x, -4.0, 60.0).astype(jnp.int32)
                yi = jnp.clip(y, -4.0, 60.0).astype(jnp.int32)
                fx = xi.astype(jnp.float32)
                fy = yi.astype(jnp.float32)
                tx = x - fx
                ty = y - fy
                ox = 1.0 - tx
                oy = 1.0 - ty
                mx0 = (xi >= 0) & (xi < W)
                mx1 = (xi >= -1) & (xi < W - 1)
                my0 = (yi >= 0) & (yi < H)
                my1 = (yi >= -1) & (yi < H - 1)
                zero = jnp.zeros((16,), jnp.float32)
                wtl = jnp.where(mx0 & my0, wkv * (ox * oy), zero)
                wtr = jnp.where(mx1 & my0, wkv * (tx * oy), zero)
                wbl = jnp.where(mx0 & my1, wkv * (ox * ty), zero)
                wbr = jnp.where(mx1 & my1, wkv * (tx * ty), zero)
                x0 = jnp.clip(xi, 0, W - 1)
                x1 = jnp.clip(xi + 1, 0, W - 1)
                y0 = jnp.clip(yi, 0, H - 1) * W
                y1 = jnp.clip(yi + 1, 0, H - 1) * W
                a00 = y0 + x0
                a10 = y0 + x1
                a01 = y1 + x0
                a11 = y1 + x1
                new = []
                for c in range(C):
                    v00 = plsc.load_gather(slab, [a00 + c * (H * W)])
                    v10 = plsc.load_gather(slab, [a10 + c * (H * W)])
                    v01 = plsc.load_gather(slab, [a01 + c * (H * W)])
                    v11 = plsc.load_gather(slab, [a11 + c * (H * W)])
                    new.append(accs[c] + (v00 * wtl + v10 * wtr
                                          + v01 * wbl + v11 * wbr))
                accs = tuple(new)

            oidx = obase + cb16 + iota
            omask = (cb16 + iota) < W
            for c in range(C):
                plsc.store_scatter(out_v, [oidx + c * W], accs[c], mask=omask)
            return 0

        lax.fori_loop(0, NCHUNK, chunk_body, 0)

        pltpu.async_copy(out_v.at[pl.ds(obase, OUT_ROW)],
                         out_hbm.at[wid, h], sem_out)
        return 0

    lax.fori_loop(0, H, row_step, 0)

    pltpu.make_async_copy(out_v.at[pl.ds(0, OUT_ROW)],
                          out_hbm.at[wid, H - 2], sem_out).wait()
    pltpu.make_async_copy(out_v.at[pl.ds(OUT_ROW, OUT_ROW)],
                          out_hbm.at[wid, H - 1], sem_out).wait()


@jax.jit
def kernel(input, deformable, weights):
    inp_t = input.transpose(0, 3, 4, 1, 2).reshape(NW, INP_WORDS)
    dx = deformable[..., 0].transpose(0, 3, 1, 4, 2)
    dy = deformable[..., 1].transpose(0, 3, 1, 4, 2)
    wk = weights.transpose(0, 3, 1, 4, 2)
    par = jnp.concatenate([dx, dy, wk], axis=3)
    par = jnp.pad(par, ((0, 0), (0, 0), (0, 0), (0, 0), (0, WP - W)))
    par = par.reshape(NW, H, PAR_ROW)
    par = jnp.pad(par, ((0, 0), (0, 0), (0, PAR_PAD - PAR_ROW)))


    mesh = plsc.VectorSubcoreMesh(core_axis_name="c", subcore_axis_name="s",
                                  num_cores=2, num_subcores=16)
    run = pl.kernel(
        _dcn_body,
        out_type=jax.ShapeDtypeStruct((NW, H, OUT_ROW), jnp.float32),
        mesh=mesh,
        scratch_types=[
            pltpu.VMEM((H * W * C,), jnp.float32),
            pltpu.VMEM((2 * PAR_PAD,), jnp.float32),
            pltpu.VMEM((2 * OUT_ROW,), jnp.float32),
            pltpu.SemaphoreType.DMA,
            pltpu.SemaphoreType.DMA,
            pltpu.SemaphoreType.DMA,
        ],
        compiler_params=pltpu.CompilerParams(needs_layout_passes=False),
    )
    out = run(inp_t, par)
    out = out.reshape(B, G, H, C, W)
    return out.transpose(0, 2, 4, 1, 3)
